# baseline (speedup 1.0000x reference)
import functools

import jax
import jax.numpy as jnp
from jax import lax
from jax.experimental import pallas as pl
from jax.experimental.pallas import tpu as pltpu
from jax.experimental.pallas import tpu_sc as plsc

_B = 16384
_D = 64
_NC = 2
_NS = 16
_NW = _NC * _NS
_BPW = _B // _NW
_CHUNK = 128
_NCHUNK = _BPW // _CHUNK

_mesh = plsc.VectorSubcoreMesh(core_axis_name="c", subcore_axis_name="s")


@functools.partial(
    pl.kernel,
    mesh=_mesh,
    out_type=jax.ShapeDtypeStruct((_B, _D), jnp.float32),
    scratch_types=[
        pltpu.VMEM((_NCHUNK, _CHUNK), jnp.int32),
        pltpu.VMEM((_BPW, _D), jnp.float32),
        pltpu.SemaphoreType.DMA,
    ],
    compiler_params=pltpu.CompilerParams(use_tc_tiling_on_sc=False),
)
def _sc_gather(idx_hbm, table_hbm, out_hbm, idx_v, rows_v, sem):
    wid = lax.axis_index("s") * _NC + lax.axis_index("c")
    base = wid * _BPW
    pltpu.sync_copy(idx_hbm.at[wid], idx_v)
    copies = [
        pltpu.async_copy(
            table_hbm.at[idx_v.at[j]],
            rows_v.at[pl.ds(j * _CHUNK, _CHUNK)],
            sem,
        )
        for j in range(_NCHUNK)
    ]
    for c in copies:
        c.wait()
    pltpu.sync_copy(rows_v, out_hbm.at[pl.ds(base, _BPW)])


def kernel(indexes, table):
    idx = indexes.reshape(_NW, _NCHUNK, _CHUNK)
    return _sc_gather(idx, table)


# speedup vs baseline: 3.5541x; 3.5541x over previous
import functools

import jax
import jax.numpy as jnp
from jax import lax
from jax.experimental import pallas as pl
from jax.experimental.pallas import tpu as pltpu
from jax.experimental.pallas import tpu_sc as plsc

_B = 16384
_D = 64
_R = 1000000
_NW = 32
_L = 16
_CW = 512
_NCH = 61
_SPAN = _NCH * _CW
_EXTRA_BASE = _NW * _SPAN
_TAIL_IN = _R - 128
_POSB = 14
_RING = 8

_mesh = plsc.VectorSubcoreMesh(core_axis_name="c", subcore_axis_name="s")


@functools.partial(
    pl.kernel,
    mesh=_mesh,
    out_type=jax.ShapeDtypeStruct((_B * _D,), jnp.float32),
    scratch_types=[
        pltpu.VMEM((_B,), jnp.int32),
        pltpu.VMEM((_B,), jnp.int32),
        pltpu.VMEM((_D, _CW), jnp.float32),
        pltpu.VMEM((_D, _CW), jnp.float32),
        pltpu.VMEM((_D, 128), jnp.float32),
        pltpu.VMEM((_L,), jnp.int32),
        pltpu.VMEM((_RING * _D,), jnp.float32),
        pltpu.SemaphoreType.DMA,
        pltpu.SemaphoreType.DMA,
        pltpu.SemaphoreType.DMA,
    ],
    compiler_params=pltpu.CompilerParams(
        use_tc_tiling_on_sc=True, needs_layout_passes=False),
)
def _sc_stream(idx_hbm, tt_hbm, tail_hbm, out_hbm,
               idx_v, match_v, c0, c1, tail_v, stage_v, ring_v,
               sem0, sem1, semc):
    wid = lax.axis_index("s") * 2 + lax.axis_index("c")
    lanes = jnp.arange(_L, dtype=jnp.int32)
    lo = wid * _SPAN
    cbufs = (c0, c1)
    csems = (sem0, sem1)

    def popcount(m):
        p = plsc.all_reduce_population_count(m)
        if p.ndim:
            p = lax.reduce_max(p, axes=(0,))
        return p

    pltpu.sync_copy(idx_hbm, idx_v)

    hi = jnp.where(wid == _NW - 1, _R, lo + _SPAN)

    def scan_body(v, cnt):
        rvec = idx_v[pl.ds(v * _L, _L)]
        m = (rvec >= lo) & (rvec < hi)
        pv = ((rvec - lo) << _POSB) | (v * _L + lanes)
        plsc.store_compressed(match_v.at[pl.ds(cnt, _L)], pv, mask=m)
        return cnt + popcount(m)

    n_match = lax.fori_loop(0, _B // _L, scan_body, jnp.int32(0))
    nvec = (n_match + _L - 1) // _L

    def fire(c, base):
        return pltpu.async_copy(
            tt_hbm.at[:, pl.ds(pl.multiple_of(base, _CW), _CW)],
            cbufs[c % 2], csems[c % 2])

    def wait_chunk(c):
        pltpu.make_async_copy(
            tt_hbm.at[:, pl.ds(0, _CW)], cbufs[c % 2], csems[c % 2]).wait()

    def process(cb, filt_lo, filt_hi, col_base, ka):
        plo = filt_lo << _POSB
        phi = filt_hi << _POSB

        def act_body(e, ka):
            svec = stage_v[...]
            p = lax.reduce_sum(jnp.where(lanes == e, svec, 0), axes=(0,))
            col = (p >> _POSB) - col_base
            pos = p & ((1 << _POSB) - 1)
            slot = ka & (_RING - 1)

            @pl.when(ka >= _RING)
            def _():
                pltpu.make_async_copy(
                    ring_v.at[pl.ds(0, _D)], out_hbm.at[pl.ds(0, _D)],
                    semc).wait()

            colvec = jnp.full((_L,), col, jnp.int32)
            base_w = slot * _D
            for g in range(_D // _L):
                vals = plsc.load_gather(cb.at[:, :], [g * _L + lanes, colvec])
                plsc.store_scatter(ring_v.at[pl.ds(0, _RING * _D)], [base_w + g * _L + lanes], vals)
            pltpu.async_copy(
                ring_v.at[pl.ds(base_w, _D)],
                out_hbm.at[pl.ds(pos * _D, _D)], semc)
            return ka + 1

        def mscan_body(v, ka):
            pvec = match_v[pl.ds(v * _L, _L)]
            valid = (v * _L + lanes) < n_match
            m = (pvec >= plo) & (pvec < phi) & valid
            plsc.store_compressed(stage_v.at[pl.ds(0, _L)], pvec, mask=m)
            return lax.fori_loop(0, popcount(m), act_body, ka)

        return lax.fori_loop(0, nvec, mscan_body, ka)

    def drain(k):
        def body(i, c):
            pltpu.make_async_copy(
                ring_v.at[pl.ds(0, _D)], out_hbm.at[pl.ds(0, _D)],
                semc).wait()
            return c

        lax.fori_loop(0, jnp.minimum(k, _RING), body, jnp.int32(0))

    fire(0, lo)
    ka = jnp.int32(0)
    for c in range(_NCH):
        if c + 1 < _NCH:
            fire(c + 1, lo + (c + 1) * _CW)
        wait_chunk(c)
        ka = process(cbufs[c % 2], c * _CW, (c + 1) * _CW, c * _CW, ka)

    @pl.when(wid == _NW - 1)
    def _():
        cp = pltpu.async_copy(
            tt_hbm.at[:, pl.ds(_EXTRA_BASE, _CW)], c0, sem0)
        tp = pltpu.async_copy(tail_hbm, tail_v, sem1)
        cp.wait()
        ka1 = process(c0, _NCH * _CW, _NCH * _CW + _CW,
                      _NCH * _CW, ka)
        tp.wait()
        ka2 = process(tail_v, _NCH * _CW + _CW, _R - _NW * _SPAN + _NCH * _CW,
                      _TAIL_IN - _EXTRA_BASE + _NCH * _CW, ka1)
        drain(ka2)

    @pl.when(wid != _NW - 1)
    def _():
        drain(ka)


def kernel(indexes, table):
    idx = indexes.reshape(_B)
    tt = table.T
    tail = lax.slice(table, (_TAIL_IN, 0), (_R, _D)).T
    flat = _sc_stream(idx, tt, tail)
    return flat.reshape(_B, _D)
